# Initial kernel scaffold; baseline (speedup 1.0000x reference)
import jax
import jax.numpy as jnp
from jax.experimental import pallas as pl


def kernel(nodes, edges, receivers, senders, globals_, n_node, n_edge, W_e1, b_e1, W_e2, b_e2, W_n1, b_n1, W_n2, b_n2):
    raise NotImplementedError("write your pallas kernel here")


# speedup vs baseline: 3.6633x; 3.6633x over previous
import functools

import jax
import jax.numpy as jnp
from jax import lax
from jax.experimental import pallas as pl
from jax.experimental.pallas import tpu as pltpu
from jax.experimental.pallas import tpu_sc as plsc

_N = 10000
_E = 2048
_DF = 128
_DE = 16
_DG = 8

_NC = 2
_NS = 16
_NW = _NC * _NS
_B = 2 * _E
_BPW = _B // _NW
_BPT = _E // _NS
_CH = 624
_CT = _N - _CH * _NS

@functools.cache
def _sc_kernels():
    mesh = plsc.VectorSubcoreMesh(
        core_axis_name="c", subcore_axis_name="s", num_cores=_NC)

    @functools.partial(
        pl.kernel,
        out_type=jax.ShapeDtypeStruct((_B, _DF), jnp.float32),
        mesh=mesh,
        scratch_types=[
            pltpu.VMEM((_BPW,), jnp.int32),
            pltpu.VMEM((_BPW, _DF), jnp.float32),
            pltpu.SemaphoreType.DMA,
        ],
    )
    def sc_gather(table_hbm, idx_hbm, out_hbm, idx_v, rows_v, sem):
        wid = lax.axis_index("s") * _NC + lax.axis_index("c")
        base = wid * _BPW
        pltpu.sync_copy(idx_hbm.at[pl.ds(base, _BPW)], idx_v)
        pltpu.async_copy(table_hbm.at[idx_v], rows_v, sem).wait()
        pltpu.sync_copy(rows_v, out_hbm.at[pl.ds(base, _BPW)])

    @functools.partial(
        pl.kernel,
        out_type=(
            jax.ShapeDtypeStruct((_N, _DF), jnp.float32),
            jax.ShapeDtypeStruct((_N, _DF), jnp.float32),
        ),
        mesh=mesh,
        scratch_types=[
            pltpu.VMEM_SHARED((_N, _DF), jnp.float32),
            pltpu.VMEM((_BPT,), jnp.int32),
            pltpu.VMEM((_BPT, _DF), jnp.float32),
        ],
    )
    def sc_scatter(basehalf_hbm, rows_hbm, idx_hbm, out0_hbm, out1_hbm,
                   acc_sh, idx_v, rows_v):
        cid = lax.axis_index("c")
        sid = lax.axis_index("s")
        coff = sid * _CH
        roff = cid * _E + sid * _BPT
        pltpu.sync_copy(idx_hbm.at[pl.ds(roff, _BPT)], idx_v)
        pltpu.sync_copy(rows_hbm.at[pl.ds(roff, _BPT)], rows_v)
        pltpu.sync_copy(basehalf_hbm.at[pl.ds(coff, _CH)],
                        acc_sh.at[pl.ds(coff, _CH)])

        @pl.when(sid == _NS - 1)
        def _():
            pltpu.sync_copy(basehalf_hbm.at[pl.ds(_CH * _NS, _CT)],
                            acc_sh.at[pl.ds(_CH * _NS, _CT)])

        plsc.subcore_barrier()
        pltpu.sync_copy(rows_v, acc_sh.at[idx_v], add=True)
        plsc.subcore_barrier()

        @pl.when(cid == 0)
        def _():
            pltpu.sync_copy(acc_sh.at[pl.ds(coff, _CH)],
                            out0_hbm.at[pl.ds(coff, _CH)])

            @pl.when(sid == _NS - 1)
            def _():
                pltpu.sync_copy(acc_sh.at[pl.ds(_CH * _NS, _CT)],
                                out0_hbm.at[pl.ds(_CH * _NS, _CT)])

        @pl.when(cid == 1)
        def _():
            pltpu.sync_copy(acc_sh.at[pl.ds(coff, _CH)],
                            out1_hbm.at[pl.ds(coff, _CH)])

            @pl.when(sid == _NS - 1)
            def _():
                pltpu.sync_copy(acc_sh.at[pl.ds(_CH * _NS, _CT)],
                                out1_hbm.at[pl.ds(_CH * _NS, _CT)])

    return sc_gather, sc_scatter


_EB = 512


def _e1_body(edges_ref, sent_ref, recv_ref, g_ref, we_ref, ws_ref, wr_ref,
             wg_ref, b_ref, out_ref):
    acc = jnp.dot(edges_ref[...], we_ref[...], preferred_element_type=jnp.float32)
    acc += jnp.dot(sent_ref[...], ws_ref[...], preferred_element_type=jnp.float32)
    acc += jnp.dot(recv_ref[...], wr_ref[...], preferred_element_type=jnp.float32)
    acc += jnp.dot(g_ref[...], wg_ref[...], preferred_element_type=jnp.float32)
    out_ref[...] = jnp.maximum(acc + b_ref[...], 0.0)


def _edge_layer1(edges, gathered, globals_, w1e, w1s, w1r, w1g, b1):
    grid = (_E // _EB,)
    return pl.pallas_call(
        _e1_body,
        grid=grid,
        in_specs=[
            pl.BlockSpec((_E, _DE), lambda j: (0, 0)),
            pl.BlockSpec((_E, _DF), lambda j: (0, 0)),
            pl.BlockSpec((_E, _DF), lambda j: (1, 0)),
            pl.BlockSpec((1, _DG), lambda j: (0, 0)),
            pl.BlockSpec((_DE, _EB), lambda j: (0, j)),
            pl.BlockSpec((_DF, _EB), lambda j: (0, j)),
            pl.BlockSpec((_DF, _EB), lambda j: (0, j)),
            pl.BlockSpec((_DG, _EB), lambda j: (0, j)),
            pl.BlockSpec((1, _EB), lambda j: (0, j)),
        ],
        out_specs=pl.BlockSpec((_E, _EB), lambda j: (0, j)),
        out_shape=jax.ShapeDtypeStruct((_E, _E), jnp.float32),
    )(edges, gathered, gathered, globals_, w1e, w1s, w1r, w1g, b1)


def _e2_body(h1_ref, w2_ref, b2_ref, ws_ref, wr_ref,
             eout_ref, es_ref, er_ref):
    j = pl.program_id(0)
    eb = jnp.dot(h1_ref[...], w2_ref[...], preferred_element_type=jnp.float32)
    eb = jnp.maximum(eb + b2_ref[...], 0.0)
    eout_ref[...] = eb
    es_p = jnp.dot(eb, ws_ref[...], preferred_element_type=jnp.float32)
    er_p = jnp.dot(eb, wr_ref[...], preferred_element_type=jnp.float32)

    @pl.when(j == 0)
    def _():
        es_ref[...] = es_p
        er_ref[...] = er_p

    @pl.when(j > 0)
    def _():
        es_ref[...] += es_p
        er_ref[...] += er_p


def _edge_layer2(h1, w2, b2, wns, wnr):
    grid = (_E // _EB,)
    return pl.pallas_call(
        _e2_body,
        grid=grid,
        in_specs=[
            pl.BlockSpec((_E, _E), lambda j: (0, 0)),
            pl.BlockSpec((_E, _EB), lambda j: (0, j)),
            pl.BlockSpec((1, _EB), lambda j: (0, j)),
            pl.BlockSpec((_EB, _DF), lambda j: (j, 0)),
            pl.BlockSpec((_EB, _DF), lambda j: (j, 0)),
        ],
        out_specs=[
            pl.BlockSpec((_E, _EB), lambda j: (0, j)),
            pl.BlockSpec((_E, _DF), lambda j: (0, 0)),
            pl.BlockSpec((_E, _DF), lambda j: (0, 0)),
        ],
        out_shape=[
            jax.ShapeDtypeStruct((_E, _E), jnp.float32),
            jax.ShapeDtypeStruct((_E, _DF), jnp.float32),
            jax.ShapeDtypeStruct((_E, _DF), jnp.float32),
        ],
    )(h1, w2, b2, wns, wnr)


_NB = 2000


def _base_body(nodes_ref, g_ref, wn_ref, wg_ref, b_ref, out_ref):
    acc = jnp.dot(nodes_ref[...], wn_ref[...], preferred_element_type=jnp.float32)
    acc += jnp.dot(g_ref[...], wg_ref[...], preferred_element_type=jnp.float32)
    out_ref[...] = 0.5 * (acc + b_ref[...])


def _node_base(nodes, globals_, wnn, wng, bn1):
    grid = (_N // _NB,)
    return pl.pallas_call(
        _base_body,
        grid=grid,
        in_specs=[
            pl.BlockSpec((_NB, _DF), lambda i: (i, 0)),
            pl.BlockSpec((1, _DG), lambda i: (0, 0)),
            pl.BlockSpec((_DF, _DF), lambda i: (0, 0)),
            pl.BlockSpec((_DG, _DF), lambda i: (0, 0)),
            pl.BlockSpec((1, _DF), lambda i: (0, 0)),
        ],
        out_specs=pl.BlockSpec((_NB, _DF), lambda i: (i, 0)),
        out_shape=jax.ShapeDtypeStruct((_N, _DF), jnp.float32),
    )(nodes, globals_, wnn, wng, bn1)


def _node_body(p0_ref, p1_ref, w2_ref, b2_ref, out_ref):
    h = jnp.maximum(p0_ref[...] + p1_ref[...], 0.0)
    acc = jnp.dot(h, w2_ref[...], preferred_element_type=jnp.float32)
    out_ref[...] = jnp.maximum(acc + b2_ref[...], 0.0)


def _node_tail(p0, p1, wn2, bn2):
    grid = (_N // _NB,)
    return pl.pallas_call(
        _node_body,
        grid=grid,
        in_specs=[
            pl.BlockSpec((_NB, _DF), lambda i: (i, 0)),
            pl.BlockSpec((_NB, _DF), lambda i: (i, 0)),
            pl.BlockSpec((_DF, _DF), lambda i: (0, 0)),
            pl.BlockSpec((1, _DF), lambda i: (0, 0)),
        ],
        out_specs=pl.BlockSpec((_NB, _DF), lambda i: (i, 0)),
        out_shape=jax.ShapeDtypeStruct((_N, _DF), jnp.float32),
    )(p0, p1, wn2, bn2)


def kernel(nodes, edges, receivers, senders, globals_, n_node, n_edge,
           W_e1, b_e1, W_e2, b_e2, W_n1, b_n1, W_n2, b_n2):
    sc_gather, sc_scatter = _sc_kernels()
    idx = jnp.concatenate([senders, receivers])
    gathered = sc_gather(nodes, idx)

    w1e = W_e1[:_DE]
    w1s = W_e1[_DE:_DE + _DF]
    w1r = W_e1[_DE + _DF:_DE + 2 * _DF]
    w1g = W_e1[_DE + 2 * _DF:]
    h1 = _edge_layer1(edges, gathered, globals_, w1e, w1s, w1r, w1g,
                      b_e1.reshape(1, -1))

    wns = W_n1[_DF:_DF + _E]
    wnr = W_n1[_DF + _E:_DF + 2 * _E]
    edges_new, es, er = _edge_layer2(h1, W_e2, b_e2.reshape(1, -1), wns, wnr)

    basehalf = _node_base(nodes, globals_, W_n1[:_DF], W_n1[_DF + 2 * _E:],
                          b_n1.reshape(1, -1))
    rows = jnp.concatenate([es, er], axis=0)
    p0, p1 = sc_scatter(basehalf, rows, idx)
    nodes_new = _node_tail(p0, p1, W_n2, b_n2.reshape(1, -1))

    return (nodes_new, edges_new, receivers, senders, globals_, n_node, n_edge)


# speedup vs baseline: 3.7550x; 1.0250x over previous
import functools

import jax
import jax.numpy as jnp
from jax import lax
from jax.experimental import pallas as pl
from jax.experimental.pallas import tpu as pltpu
from jax.experimental.pallas import tpu_sc as plsc

_N = 10000
_E = 2048
_DF = 128
_DE = 16
_DG = 8

_NC = 2
_NS = 16
_NW = _NC * _NS
_B = 2 * _E
_BPW = _B // _NW
_BPT = _E // _NS
_CH = 624
_CT = _N - _CH * _NS

@functools.cache
def _sc_kernels():
    mesh = plsc.VectorSubcoreMesh(
        core_axis_name="c", subcore_axis_name="s", num_cores=_NC)

    @functools.partial(
        pl.kernel,
        out_type=jax.ShapeDtypeStruct((_B, _DF), jnp.float32),
        mesh=mesh,
        scratch_types=[
            pltpu.VMEM((_BPW,), jnp.int32),
            pltpu.VMEM((_BPW, _DF), jnp.float32),
            pltpu.SemaphoreType.DMA,
        ],
    )
    def sc_gather(table_hbm, idx_hbm, out_hbm, idx_v, rows_v, sem):
        wid = lax.axis_index("s") * _NC + lax.axis_index("c")
        base = wid * _BPW
        pltpu.sync_copy(idx_hbm.at[pl.ds(base, _BPW)], idx_v)
        pltpu.async_copy(table_hbm.at[idx_v], rows_v, sem).wait()
        pltpu.sync_copy(rows_v, out_hbm.at[pl.ds(base, _BPW)])

    @functools.partial(
        pl.kernel,
        out_type=(
            jax.ShapeDtypeStruct((_N, _DF), jnp.float32),
            jax.ShapeDtypeStruct((_N, _DF), jnp.float32),
        ),
        mesh=mesh,
        scratch_types=[
            pltpu.VMEM_SHARED((_N, _DF), jnp.float32),
            pltpu.VMEM((_BPT,), jnp.int32),
            pltpu.VMEM((_BPT, _DF), jnp.float32),
        ],
    )
    def sc_scatter(basehalf_hbm, rows_hbm, idx_hbm, out0_hbm, out1_hbm,
                   acc_sh, idx_v, rows_v):
        cid = lax.axis_index("c")
        sid = lax.axis_index("s")
        coff = sid * _CH
        roff = cid * _E + sid * _BPT
        pltpu.sync_copy(idx_hbm.at[pl.ds(roff, _BPT)], idx_v)
        pltpu.sync_copy(rows_hbm.at[pl.ds(roff, _BPT)], rows_v)
        pltpu.sync_copy(basehalf_hbm.at[pl.ds(coff, _CH)],
                        acc_sh.at[pl.ds(coff, _CH)])

        @pl.when(sid == _NS - 1)
        def _():
            pltpu.sync_copy(basehalf_hbm.at[pl.ds(_CH * _NS, _CT)],
                            acc_sh.at[pl.ds(_CH * _NS, _CT)])

        plsc.subcore_barrier()
        pltpu.sync_copy(rows_v, acc_sh.at[idx_v], add=True)
        plsc.subcore_barrier()

        @pl.when(cid == 0)
        def _():
            pltpu.sync_copy(acc_sh.at[pl.ds(coff, _CH)],
                            out0_hbm.at[pl.ds(coff, _CH)])

            @pl.when(sid == _NS - 1)
            def _():
                pltpu.sync_copy(acc_sh.at[pl.ds(_CH * _NS, _CT)],
                                out0_hbm.at[pl.ds(_CH * _NS, _CT)])

        @pl.when(cid == 1)
        def _():
            pltpu.sync_copy(acc_sh.at[pl.ds(coff, _CH)],
                            out1_hbm.at[pl.ds(coff, _CH)])

            @pl.when(sid == _NS - 1)
            def _():
                pltpu.sync_copy(acc_sh.at[pl.ds(_CH * _NS, _CT)],
                                out1_hbm.at[pl.ds(_CH * _NS, _CT)])

    return sc_gather, sc_scatter


_EB = 512


def _bf(x):
    return x.astype(jnp.bfloat16)


def _e1_body(edges_ref, sent_ref, recv_ref, g_ref, we_ref, ws_ref, wr_ref,
             wg_ref, b_ref, out_ref):
    acc = jnp.dot(_bf(edges_ref[...]), _bf(we_ref[...]),
                  preferred_element_type=jnp.float32)
    acc += jnp.dot(_bf(sent_ref[...]), _bf(ws_ref[...]),
                   preferred_element_type=jnp.float32)
    acc += jnp.dot(_bf(recv_ref[...]), _bf(wr_ref[...]),
                   preferred_element_type=jnp.float32)
    acc += jnp.dot(g_ref[...], wg_ref[...], preferred_element_type=jnp.float32)
    out_ref[...] = _bf(jnp.maximum(acc + b_ref[...], 0.0))


def _edge_layer1(edges, gathered, globals_, w1e, w1s, w1r, w1g, b1):
    grid = (_E // _EB,)
    return pl.pallas_call(
        _e1_body,
        grid=grid,
        in_specs=[
            pl.BlockSpec((_E, _DE), lambda j: (0, 0)),
            pl.BlockSpec((_E, _DF), lambda j: (0, 0)),
            pl.BlockSpec((_E, _DF), lambda j: (1, 0)),
            pl.BlockSpec((1, _DG), lambda j: (0, 0)),
            pl.BlockSpec((_DE, _EB), lambda j: (0, j)),
            pl.BlockSpec((_DF, _EB), lambda j: (0, j)),
            pl.BlockSpec((_DF, _EB), lambda j: (0, j)),
            pl.BlockSpec((_DG, _EB), lambda j: (0, j)),
            pl.BlockSpec((1, _EB), lambda j: (0, j)),
        ],
        out_specs=pl.BlockSpec((_E, _EB), lambda j: (0, j)),
        out_shape=jax.ShapeDtypeStruct((_E, _E), jnp.bfloat16),
    )(edges, gathered, gathered, globals_, w1e, w1s, w1r, w1g, b1)


def _e2_body(h1_ref, w2_ref, b2_ref, ws_ref, wr_ref,
             eout_ref, es_ref, er_ref):
    j = pl.program_id(0)
    eb = jnp.dot(h1_ref[...], _bf(w2_ref[...]),
                 preferred_element_type=jnp.float32)
    eb = jnp.maximum(eb + b2_ref[...], 0.0)
    eout_ref[...] = eb
    ebb = _bf(eb)
    es_p = jnp.dot(ebb, _bf(ws_ref[...]), preferred_element_type=jnp.float32)
    er_p = jnp.dot(ebb, _bf(wr_ref[...]), preferred_element_type=jnp.float32)

    @pl.when(j == 0)
    def _():
        es_ref[...] = es_p
        er_ref[...] = er_p

    @pl.when(j > 0)
    def _():
        es_ref[...] += es_p
        er_ref[...] += er_p


def _edge_layer2(h1, w2, b2, wns, wnr):
    grid = (_E // _EB,)
    return pl.pallas_call(
        _e2_body,
        grid=grid,
        in_specs=[
            pl.BlockSpec((_E, _E), lambda j: (0, 0)),
            pl.BlockSpec((_E, _EB), lambda j: (0, j)),
            pl.BlockSpec((1, _EB), lambda j: (0, j)),
            pl.BlockSpec((_EB, _DF), lambda j: (j, 0)),
            pl.BlockSpec((_EB, _DF), lambda j: (j, 0)),
        ],
        out_specs=[
            pl.BlockSpec((_E, _EB), lambda j: (0, j)),
            pl.BlockSpec((_E, _DF), lambda j: (0, 0)),
            pl.BlockSpec((_E, _DF), lambda j: (0, 0)),
        ],
        out_shape=[
            jax.ShapeDtypeStruct((_E, _E), jnp.float32),
            jax.ShapeDtypeStruct((_E, _DF), jnp.float32),
            jax.ShapeDtypeStruct((_E, _DF), jnp.float32),
        ],
    )(h1, w2, b2, wns, wnr)


_NB = 2000


def _base_body(nodes_ref, g_ref, wn_ref, wg_ref, b_ref, out_ref):
    acc = jnp.dot(_bf(nodes_ref[...]), _bf(wn_ref[...]),
                  preferred_element_type=jnp.float32)
    acc += jnp.dot(g_ref[...], wg_ref[...], preferred_element_type=jnp.float32)
    out_ref[...] = 0.5 * (acc + b_ref[...])


def _node_base(nodes, globals_, wnn, wng, bn1):
    grid = (_N // _NB,)
    return pl.pallas_call(
        _base_body,
        grid=grid,
        in_specs=[
            pl.BlockSpec((_NB, _DF), lambda i: (i, 0)),
            pl.BlockSpec((1, _DG), lambda i: (0, 0)),
            pl.BlockSpec((_DF, _DF), lambda i: (0, 0)),
            pl.BlockSpec((_DG, _DF), lambda i: (0, 0)),
            pl.BlockSpec((1, _DF), lambda i: (0, 0)),
        ],
        out_specs=pl.BlockSpec((_NB, _DF), lambda i: (i, 0)),
        out_shape=jax.ShapeDtypeStruct((_N, _DF), jnp.float32),
    )(nodes, globals_, wnn, wng, bn1)


def _node_body(p0_ref, p1_ref, w2_ref, b2_ref, out_ref):
    h = jnp.maximum(p0_ref[...] + p1_ref[...], 0.0)
    acc = jnp.dot(_bf(h), _bf(w2_ref[...]), preferred_element_type=jnp.float32)
    out_ref[...] = jnp.maximum(acc + b2_ref[...], 0.0)


def _node_tail(p0, p1, wn2, bn2):
    grid = (_N // _NB,)
    return pl.pallas_call(
        _node_body,
        grid=grid,
        in_specs=[
            pl.BlockSpec((_NB, _DF), lambda i: (i, 0)),
            pl.BlockSpec((_NB, _DF), lambda i: (i, 0)),
            pl.BlockSpec((_DF, _DF), lambda i: (0, 0)),
            pl.BlockSpec((1, _DF), lambda i: (0, 0)),
        ],
        out_specs=pl.BlockSpec((_NB, _DF), lambda i: (i, 0)),
        out_shape=jax.ShapeDtypeStruct((_N, _DF), jnp.float32),
    )(p0, p1, wn2, bn2)


def kernel(nodes, edges, receivers, senders, globals_, n_node, n_edge,
           W_e1, b_e1, W_e2, b_e2, W_n1, b_n1, W_n2, b_n2):
    sc_gather, sc_scatter = _sc_kernels()
    idx = jnp.concatenate([senders, receivers])
    gathered = sc_gather(nodes, idx)

    w1e = W_e1[:_DE]
    w1s = W_e1[_DE:_DE + _DF]
    w1r = W_e1[_DE + _DF:_DE + 2 * _DF]
    w1g = W_e1[_DE + 2 * _DF:]
    h1 = _edge_layer1(edges, gathered, globals_, w1e, w1s, w1r, w1g,
                      b_e1.reshape(1, -1))

    wns = W_n1[_DF:_DF + _E]
    wnr = W_n1[_DF + _E:_DF + 2 * _E]
    edges_new, es, er = _edge_layer2(h1, W_e2, b_e2.reshape(1, -1), wns, wnr)

    basehalf = _node_base(nodes, globals_, W_n1[:_DF], W_n1[_DF + 2 * _E:],
                          b_n1.reshape(1, -1))
    rows = jnp.concatenate([es, er], axis=0)
    p0, p1 = sc_scatter(basehalf, rows, idx)
    nodes_new = _node_tail(p0, p1, W_n2, b_n2.reshape(1, -1))

    return (nodes_new, edges_new, receivers, senders, globals_, n_node, n_edge)


# speedup vs baseline: 4.1951x; 1.1172x over previous
import functools

import jax
import jax.numpy as jnp
from jax import lax
from jax.experimental import pallas as pl
from jax.experimental.pallas import tpu as pltpu
from jax.experimental.pallas import tpu_sc as plsc

_N = 10000
_E = 2048
_DF = 128
_DE = 16
_DG = 8

_NC = 2
_NS = 16
_NW = _NC * _NS
_B = 2 * _E
_BPW = _B // _NW
_BPT = _E // _NS
_CH = 624
_CT = _N - _CH * _NS

@functools.cache
def _sc_kernels():
    mesh = plsc.VectorSubcoreMesh(
        core_axis_name="c", subcore_axis_name="s", num_cores=_NC)

    @functools.partial(
        pl.kernel,
        out_type=jax.ShapeDtypeStruct((_B, _DF), jnp.float32),
        mesh=mesh,
        scratch_types=[
            pltpu.VMEM((_BPW,), jnp.int32),
            pltpu.VMEM((_BPW, _DF), jnp.float32),
            pltpu.SemaphoreType.DMA,
        ],
    )
    def sc_gather(table_hbm, idx_hbm, out_hbm, idx_v, rows_v, sem):
        wid = lax.axis_index("s") * _NC + lax.axis_index("c")
        base = wid * _BPW
        pltpu.sync_copy(idx_hbm.at[pl.ds(base, _BPW)], idx_v)
        pltpu.async_copy(table_hbm.at[idx_v], rows_v, sem).wait()
        pltpu.sync_copy(rows_v, out_hbm.at[pl.ds(base, _BPW)])

    @functools.partial(
        pl.kernel,
        out_type=(
            jax.ShapeDtypeStruct((_N, _DF), jnp.float32),
            jax.ShapeDtypeStruct((_N, _DF), jnp.float32),
        ),
        mesh=mesh,
        scratch_types=[
            pltpu.VMEM_SHARED((_N, _DF), jnp.float32),
            pltpu.VMEM((_BPT,), jnp.int32),
            pltpu.VMEM((_BPT, _DF), jnp.float32),
        ],
    )
    def sc_scatter(basehalf_hbm, rows_hbm, idx_hbm, out0_hbm, out1_hbm,
                   acc_sh, idx_v, rows_v):
        cid = lax.axis_index("c")
        sid = lax.axis_index("s")
        coff = sid * _CH
        roff = cid * _E + sid * _BPT
        pltpu.sync_copy(idx_hbm.at[pl.ds(roff, _BPT)], idx_v)
        pltpu.sync_copy(rows_hbm.at[pl.ds(roff, _BPT)], rows_v)
        pltpu.sync_copy(basehalf_hbm.at[pl.ds(coff, _CH)],
                        acc_sh.at[pl.ds(coff, _CH)])

        @pl.when(sid == _NS - 1)
        def _():
            pltpu.sync_copy(basehalf_hbm.at[pl.ds(_CH * _NS, _CT)],
                            acc_sh.at[pl.ds(_CH * _NS, _CT)])

        plsc.subcore_barrier()
        pltpu.sync_copy(rows_v, acc_sh.at[idx_v], add=True)
        plsc.subcore_barrier()

        @pl.when(cid == 0)
        def _():
            pltpu.sync_copy(acc_sh.at[pl.ds(coff, _CH)],
                            out0_hbm.at[pl.ds(coff, _CH)])

            @pl.when(sid == _NS - 1)
            def _():
                pltpu.sync_copy(acc_sh.at[pl.ds(_CH * _NS, _CT)],
                                out0_hbm.at[pl.ds(_CH * _NS, _CT)])

        @pl.when(cid == 1)
        def _():
            pltpu.sync_copy(acc_sh.at[pl.ds(coff, _CH)],
                            out1_hbm.at[pl.ds(coff, _CH)])

            @pl.when(sid == _NS - 1)
            def _():
                pltpu.sync_copy(acc_sh.at[pl.ds(_CH * _NS, _CT)],
                                out1_hbm.at[pl.ds(_CH * _NS, _CT)])

    return sc_gather, sc_scatter


_EB = 512


def _bf(x):
    return x.astype(jnp.bfloat16)


_KB = _E // _EB
_NB = 2000
_NBK = _N // _NB


def _fused_body(edges_ref, sent_ref, recv_ref, g_ref,
                w1e_ref, w1s_ref, w1r_ref, w1g_ref, b1_ref,
                w2_ref, b2_ref, wns_ref, wnr_ref,
                nodes_ref, wnn_ref, wng_ref, bn1_ref,
                eout_ref, rows_ref, base_ref, h1_scr):
    j = pl.program_id(0)

    @pl.when(j < _KB)
    def _():
        acc = jnp.dot(_bf(edges_ref[...]), _bf(w1e_ref[...]),
                      preferred_element_type=jnp.float32)
        acc += jnp.dot(_bf(sent_ref[...]), _bf(w1s_ref[...]),
                       preferred_element_type=jnp.float32)
        acc += jnp.dot(_bf(recv_ref[...]), _bf(w1r_ref[...]),
                       preferred_element_type=jnp.float32)
        acc += jnp.dot(g_ref[...], w1g_ref[...],
                       preferred_element_type=jnp.float32)
        h1_scr[j] = _bf(jnp.maximum(acc + b1_ref[...], 0.0))

    @pl.when(jnp.logical_and(j >= _KB, j < 2 * _KB))
    def _():
        acc = jnp.zeros((_E, _EB), jnp.float32)
        for k in range(_KB):
            acc += jnp.dot(h1_scr[k], _bf(w2_ref[pl.ds(k * _EB, _EB), :]),
                           preferred_element_type=jnp.float32)
        eb = jnp.maximum(acc + b2_ref[...], 0.0)
        eout_ref[...] = eb
        ebb = _bf(eb)
        es_p = jnp.dot(ebb, _bf(wns_ref[...]), preferred_element_type=jnp.float32)
        er_p = jnp.dot(ebb, _bf(wnr_ref[...]), preferred_element_type=jnp.float32)

        @pl.when(j == _KB)
        def _():
            rows_ref[:_E] = es_p
            rows_ref[_E:] = er_p

        @pl.when(j > _KB)
        def _():
            rows_ref[:_E] += es_p
            rows_ref[_E:] += er_p

    @pl.when(j >= 2 * _KB)
    def _():
        acc = jnp.dot(_bf(nodes_ref[...]), _bf(wnn_ref[...]),
                      preferred_element_type=jnp.float32)
        acc += jnp.dot(g_ref[...], wng_ref[...],
                       preferred_element_type=jnp.float32)
        base_ref[...] = 0.5 * (acc + bn1_ref[...])


def _fused_tc(edges, gathered, globals_, w1e, w1s, w1r, w1g, b1,
              w2, b2, wns, wnr, nodes, wnn, wng, bn1):
    def _jb(j):
        return jnp.clip(j - _KB, 0, _KB - 1)

    def _jn(j):
        return jnp.clip(j - 2 * _KB, 0, _NBK - 1)

    grid = (2 * _KB + _NBK,)
    return pl.pallas_call(
        _fused_body,
        grid=grid,
        in_specs=[
            pl.BlockSpec((_E, _DE), lambda j: (0, 0)),
            pl.BlockSpec((_E, _DF), lambda j: (0, 0)),
            pl.BlockSpec((_E, _DF), lambda j: (1, 0)),
            pl.BlockSpec((1, _DG), lambda j: (0, 0)),
            pl.BlockSpec((_DE, _EB), lambda j: (0, jnp.minimum(j, _KB - 1))),
            pl.BlockSpec((_DF, _EB), lambda j: (0, jnp.minimum(j, _KB - 1))),
            pl.BlockSpec((_DF, _EB), lambda j: (0, jnp.minimum(j, _KB - 1))),
            pl.BlockSpec((_DG, _EB), lambda j: (0, jnp.minimum(j, _KB - 1))),
            pl.BlockSpec((1, _EB), lambda j: (0, jnp.minimum(j, _KB - 1))),
            pl.BlockSpec((_E, _EB), lambda j: (0, _jb(j))),
            pl.BlockSpec((1, _EB), lambda j: (0, _jb(j))),
            pl.BlockSpec((_EB, _DF), lambda j: (_jb(j), 0)),
            pl.BlockSpec((_EB, _DF), lambda j: (_jb(j), 0)),
            pl.BlockSpec((_NB, _DF), lambda j: (_jn(j), 0)),
            pl.BlockSpec((_DF, _DF), lambda j: (0, 0)),
            pl.BlockSpec((_DG, _DF), lambda j: (0, 0)),
            pl.BlockSpec((1, _DF), lambda j: (0, 0)),
        ],
        out_specs=[
            pl.BlockSpec((_E, _EB), lambda j: (0, _jb(j))),
            pl.BlockSpec((2 * _E, _DF), lambda j: (0, 0)),
            pl.BlockSpec((_NB, _DF), lambda j: (_jn(j), 0)),
        ],
        out_shape=[
            jax.ShapeDtypeStruct((_E, _E), jnp.float32),
            jax.ShapeDtypeStruct((2 * _E, _DF), jnp.float32),
            jax.ShapeDtypeStruct((_N, _DF), jnp.float32),
        ],
        scratch_shapes=[pltpu.VMEM((_KB, _E, _EB), jnp.bfloat16)],
    )(edges, gathered, gathered, globals_, w1e, w1s, w1r, w1g, b1,
      w2, b2, wns, wnr, nodes, wnn, wng, bn1)


def _node_body(p0_ref, p1_ref, w2_ref, b2_ref, out_ref):
    h = jnp.maximum(p0_ref[...] + p1_ref[...], 0.0)
    acc = jnp.dot(_bf(h), _bf(w2_ref[...]), preferred_element_type=jnp.float32)
    out_ref[...] = jnp.maximum(acc + b2_ref[...], 0.0)


def _node_tail(p0, p1, wn2, bn2):
    grid = (_N // _NB,)
    return pl.pallas_call(
        _node_body,
        grid=grid,
        in_specs=[
            pl.BlockSpec((_NB, _DF), lambda i: (i, 0)),
            pl.BlockSpec((_NB, _DF), lambda i: (i, 0)),
            pl.BlockSpec((_DF, _DF), lambda i: (0, 0)),
            pl.BlockSpec((1, _DF), lambda i: (0, 0)),
        ],
        out_specs=pl.BlockSpec((_NB, _DF), lambda i: (i, 0)),
        out_shape=jax.ShapeDtypeStruct((_N, _DF), jnp.float32),
    )(p0, p1, wn2, bn2)


def kernel(nodes, edges, receivers, senders, globals_, n_node, n_edge,
           W_e1, b_e1, W_e2, b_e2, W_n1, b_n1, W_n2, b_n2):
    sc_gather, sc_scatter = _sc_kernels()
    idx = jnp.concatenate([senders, receivers])
    gathered = sc_gather(nodes, idx)

    w1e = W_e1[:_DE]
    w1s = W_e1[_DE:_DE + _DF]
    w1r = W_e1[_DE + _DF:_DE + 2 * _DF]
    w1g = W_e1[_DE + 2 * _DF:]
    wns = W_n1[_DF:_DF + _E]
    wnr = W_n1[_DF + _E:_DF + 2 * _E]
    edges_new, rows, basehalf = _fused_tc(
        edges, gathered, globals_, w1e, w1s, w1r, w1g, b_e1.reshape(1, -1),
        W_e2, b_e2.reshape(1, -1), wns, wnr,
        nodes, W_n1[:_DF], W_n1[_DF + 2 * _E:], b_n1.reshape(1, -1))
    p0, p1 = sc_scatter(basehalf, rows, idx)
    nodes_new = _node_tail(p0, p1, W_n2, b_n2.reshape(1, -1))

    return (nodes_new, edges_new, receivers, senders, globals_, n_node, n_edge)


# speedup vs baseline: 4.3341x; 1.0331x over previous
import functools

import jax
import jax.numpy as jnp
from jax import lax
from jax.experimental import pallas as pl
from jax.experimental.pallas import tpu as pltpu
from jax.experimental.pallas import tpu_sc as plsc

_N = 10000
_E = 2048
_DF = 128
_DE = 16
_DG = 8

_NC = 2
_NS = 16
_NW = _NC * _NS
_B = 2 * _E
_BPW = _B // _NW
_BPT = _E // _NS
_CH = 624
_CT = _N - _CH * _NS

@functools.cache
def _sc_kernels():
    mesh = plsc.VectorSubcoreMesh(
        core_axis_name="c", subcore_axis_name="s", num_cores=_NC)

    @functools.partial(
        pl.kernel,
        out_type=jax.ShapeDtypeStruct((_B, _DF), jnp.float32),
        mesh=mesh,
        scratch_types=[
            pltpu.VMEM((_BPW,), jnp.int32),
            pltpu.VMEM((_BPW, _DF), jnp.float32),
            pltpu.SemaphoreType.DMA,
        ],
    )
    def sc_gather(table_hbm, idx_hbm, out_hbm, idx_v, rows_v, sem):
        wid = lax.axis_index("s") * _NC + lax.axis_index("c")
        base = wid * _BPW
        pltpu.sync_copy(idx_hbm.at[pl.ds(base, _BPW)], idx_v)
        pltpu.async_copy(table_hbm.at[idx_v], rows_v, sem).wait()
        pltpu.sync_copy(rows_v, out_hbm.at[pl.ds(base, _BPW)])

    @functools.partial(
        pl.kernel,
        out_type=(
            jax.ShapeDtypeStruct((_N, _DF), jnp.float32),
            jax.ShapeDtypeStruct((_N, _DF), jnp.float32),
        ),
        mesh=mesh,
        scratch_types=[
            pltpu.VMEM_SHARED((_N, _DF), jnp.float32),
            pltpu.VMEM((_BPT,), jnp.int32),
            pltpu.VMEM((_BPT, _DF), jnp.float32),
            pltpu.SemaphoreType.DMA,
            pltpu.SemaphoreType.DMA,
            pltpu.SemaphoreType.DMA,
        ],
    )
    def sc_scatter(basehalf_hbm, rows_hbm, idx_hbm, out0_hbm, out1_hbm,
                   acc_sh, idx_v, rows_v, sem_i, sem_r, sem_b):
        cid = lax.axis_index("c")
        sid = lax.axis_index("s")
        coff = sid * _CH
        roff = cid * _E + sid * _BPT
        c_i = pltpu.async_copy(idx_hbm.at[pl.ds(roff, _BPT)], idx_v, sem_i)
        c_r = pltpu.async_copy(rows_hbm.at[pl.ds(roff, _BPT)], rows_v, sem_r)
        c_b = pltpu.async_copy(basehalf_hbm.at[pl.ds(coff, _CH)],
                               acc_sh.at[pl.ds(coff, _CH)], sem_b)

        @pl.when(sid == _NS - 1)
        def _():
            pltpu.sync_copy(basehalf_hbm.at[pl.ds(_CH * _NS, _CT)],
                            acc_sh.at[pl.ds(_CH * _NS, _CT)])

        c_i.wait()
        c_r.wait()
        c_b.wait()
        plsc.subcore_barrier()
        pltpu.sync_copy(rows_v, acc_sh.at[idx_v], add=True)
        plsc.subcore_barrier()

        @pl.when(cid == 0)
        def _():
            pltpu.sync_copy(acc_sh.at[pl.ds(coff, _CH)],
                            out0_hbm.at[pl.ds(coff, _CH)])

            @pl.when(sid == _NS - 1)
            def _():
                pltpu.sync_copy(acc_sh.at[pl.ds(_CH * _NS, _CT)],
                                out0_hbm.at[pl.ds(_CH * _NS, _CT)])

        @pl.when(cid == 1)
        def _():
            pltpu.sync_copy(acc_sh.at[pl.ds(coff, _CH)],
                            out1_hbm.at[pl.ds(coff, _CH)])

            @pl.when(sid == _NS - 1)
            def _():
                pltpu.sync_copy(acc_sh.at[pl.ds(_CH * _NS, _CT)],
                                out1_hbm.at[pl.ds(_CH * _NS, _CT)])

    return sc_gather, sc_scatter


_EB = 512


def _bf(x):
    return x.astype(jnp.bfloat16)


_KB = _E // _EB
_NB = 2000
_NBK = _N // _NB
_IN_E = _DE + 2 * _DF + _DG


def _fused_body(ein_ref, g_ref,
                w1_ref, b1_ref,
                w2_ref, b2_ref, wns_ref, wnr_ref,
                nodes_ref, wnn_ref, wng_ref, bn1_ref,
                eout_ref, rows_ref, base_ref, h1_scr):
    j = pl.program_id(0)

    for jj in range(_KB):
        @pl.when(j == jj)
        def _():
            acc = jnp.dot(ein_ref[...], _bf(w1_ref[...]),
                          preferred_element_type=jnp.float32)
            h1_scr[:, jj * _EB:(jj + 1) * _EB] = _bf(
                jnp.maximum(acc + b1_ref[...], 0.0))

    @pl.when(jnp.logical_and(j >= _KB, j < 2 * _KB))
    def _():
        acc = jnp.dot(h1_scr[...], _bf(w2_ref[...]),
                      preferred_element_type=jnp.float32)
        eb = jnp.maximum(acc + b2_ref[...], 0.0)
        eout_ref[...] = eb
        ebb = _bf(eb)
        es_p = jnp.dot(ebb, _bf(wns_ref[...]), preferred_element_type=jnp.float32)
        er_p = jnp.dot(ebb, _bf(wnr_ref[...]), preferred_element_type=jnp.float32)

        @pl.when(j == _KB)
        def _():
            rows_ref[:_E] = es_p
            rows_ref[_E:] = er_p

        @pl.when(j > _KB)
        def _():
            rows_ref[:_E] += es_p
            rows_ref[_E:] += er_p

    @pl.when(j >= 2 * _KB)
    def _():
        acc = jnp.dot(_bf(nodes_ref[...]), _bf(wnn_ref[...]),
                      preferred_element_type=jnp.float32)
        acc += jnp.dot(g_ref[...], wng_ref[...],
                       preferred_element_type=jnp.float32)
        base_ref[...] = 0.5 * (acc + bn1_ref[...])


def _fused_tc(ein, globals_, w1, b1,
              w2, b2, wns, wnr, nodes, wnn, wng, bn1):
    def _jb(j):
        return jnp.clip(j - _KB, 0, _KB - 1)

    def _jn(j):
        return jnp.clip(j - 2 * _KB, 0, _NBK - 1)

    grid = (2 * _KB + _NBK,)
    return pl.pallas_call(
        _fused_body,
        grid=grid,
        in_specs=[
            pl.BlockSpec((_E, _IN_E), lambda j: (0, 0)),
            pl.BlockSpec((1, _DG), lambda j: (0, 0)),
            pl.BlockSpec((_IN_E, _EB), lambda j: (0, jnp.minimum(j, _KB - 1))),
            pl.BlockSpec((1, _EB), lambda j: (0, jnp.minimum(j, _KB - 1))),
            pl.BlockSpec((_E, _EB), lambda j: (0, _jb(j))),
            pl.BlockSpec((1, _EB), lambda j: (0, _jb(j))),
            pl.BlockSpec((_EB, _DF), lambda j: (_jb(j), 0)),
            pl.BlockSpec((_EB, _DF), lambda j: (_jb(j), 0)),
            pl.BlockSpec((_NB, _DF), lambda j: (_jn(j), 0)),
            pl.BlockSpec((_DF, _DF), lambda j: (0, 0)),
            pl.BlockSpec((_DG, _DF), lambda j: (0, 0)),
            pl.BlockSpec((1, _DF), lambda j: (0, 0)),
        ],
        out_specs=[
            pl.BlockSpec((_E, _EB), lambda j: (0, _jb(j))),
            pl.BlockSpec((2 * _E, _DF), lambda j: (0, 0)),
            pl.BlockSpec((_NB, _DF), lambda j: (_jn(j), 0)),
        ],
        out_shape=[
            jax.ShapeDtypeStruct((_E, _E), jnp.float32),
            jax.ShapeDtypeStruct((2 * _E, _DF), jnp.float32),
            jax.ShapeDtypeStruct((_N, _DF), jnp.float32),
        ],
        scratch_shapes=[pltpu.VMEM((_E, _E), jnp.bfloat16)],
    )(ein, globals_, w1, b1,
      w2, b2, wns, wnr, nodes, wnn, wng, bn1)


_TB = 1000


def _node_body(p0_ref, p1_ref, w2_ref, b2_ref, out_ref):
    h = jnp.maximum(p0_ref[...] + p1_ref[...], 0.0)
    acc = jnp.dot(_bf(h), _bf(w2_ref[...]), preferred_element_type=jnp.float32)
    out_ref[...] = jnp.maximum(acc + b2_ref[...], 0.0)


def _node_tail(p0, p1, wn2, bn2):
    grid = (_N // _TB,)
    return pl.pallas_call(
        _node_body,
        grid=grid,
        in_specs=[
            pl.BlockSpec((_TB, _DF), lambda i: (i, 0)),
            pl.BlockSpec((_TB, _DF), lambda i: (i, 0)),
            pl.BlockSpec((_DF, _DF), lambda i: (0, 0)),
            pl.BlockSpec((1, _DF), lambda i: (0, 0)),
        ],
        out_specs=pl.BlockSpec((_TB, _DF), lambda i: (i, 0)),
        out_shape=jax.ShapeDtypeStruct((_N, _DF), jnp.float32),
    )(p0, p1, wn2, bn2)


def kernel(nodes, edges, receivers, senders, globals_, n_node, n_edge,
           W_e1, b_e1, W_e2, b_e2, W_n1, b_n1, W_n2, b_n2):
    sc_gather, sc_scatter = _sc_kernels()
    idx = jnp.concatenate([senders, receivers])
    gathered = sc_gather(nodes, idx)

    ein = jnp.concatenate(
        [edges, gathered[:_E], gathered[_E:],
         jnp.broadcast_to(globals_[0], (_E, _DG))],
        axis=1).astype(jnp.bfloat16)
    wns = W_n1[_DF:_DF + _E]
    wnr = W_n1[_DF + _E:_DF + 2 * _E]
    edges_new, rows, basehalf = _fused_tc(
        ein, globals_, W_e1, b_e1.reshape(1, -1),
        W_e2, b_e2.reshape(1, -1), wns, wnr,
        nodes, W_n1[:_DF], W_n1[_DF + 2 * _E:], b_n1.reshape(1, -1))
    p0, p1 = sc_scatter(basehalf, rows, idx)
    nodes_new = _node_tail(p0, p1, W_n2, b_n2.reshape(1, -1))

    return (nodes_new, edges_new, receivers, senders, globals_, n_node, n_edge)
